# baseline (speedup 1.0000x reference)
import math

import jax
import jax.numpy as jnp
import numpy as np
from jax import lax
from jax.experimental import pallas as pl
from jax.experimental.pallas import tpu as pltpu
from jax.experimental.pallas import tpu_sc as plsc

_L = 16
_T = 2 ** 19
_F = 2
_N_MIN = 16
_N_MAX = 2048
_BB_MIN = -1.0
_GROWTH = math.exp((math.log(_N_MAX) - math.log(_N_MIN)) / (_L - 1))
_RES = [int(math.floor(_N_MIN * (_GROWTH ** i))) for i in range(_L)]
_CELL = [np.float32(2.0 / r) for r in _RES]
_PI1 = np.uint32(2654435761)
_PI2 = np.uint32(805459861)
_MASK = np.uint32(_T - 1)

_N = 262144
_NW = 32
_P = _N // _NW
_C = 512
_NCH = _P // _C
_E = 2 * 8 * _C


def _sc_body(x_hbm, y_hbm, z_hbm, tab_hbm, out_hbm,
             xv, yv, zv, idxv, rowsv, outv, tabs, sem):
    sid = lax.axis_index("s")
    wid = sid * 2 + lax.axis_index("c")
    lane = lax.iota(jnp.int32, 16)
    halfl = lax.shift_right_logical(lane, 1)
    feat = lane & 1
    pos_e = 2 * lane

    pbase = wid * _P
    pltpu.sync_copy(x_hbm.at[pl.ds(pbase, _P)], xv)
    pltpu.sync_copy(y_hbm.at[pl.ds(pbase, _P)], yv)
    pltpu.sync_copy(z_hbm.at[pl.ds(pbase, _P)], zv)

    for i in range(_L):
        cell = _CELL[i]

        @pl.when(sid == 0)
        def _():
            pltpu.sync_copy(tab_hbm.at[pl.ds(i * (2 * _T), 2 * _T)], tabs)

        plsc.subcore_barrier()

        def chunk_body(ch, carry):
            cp0 = ch * _C

            def idx_body(g, c2):
                col = cp0 + g * 16
                px = xv[pl.ds(col, 16)]
                py = yv[pl.ds(col, 16)]
                pz = zv[pl.ds(col, 16)]
                ux = ((px - jnp.float32(_BB_MIN)) / cell).astype(jnp.int32).astype(jnp.uint32)
                uy = ((py - jnp.float32(_BB_MIN)) / cell).astype(jnp.int32).astype(jnp.uint32)
                uz = ((pz - jnp.float32(_BB_MIN)) / cell).astype(jnp.int32).astype(jnp.uint32)
                hx = (ux, ux + np.uint32(1))
                hy = (uy * _PI1, (uy + np.uint32(1)) * _PI1)
                hz = (uz * _PI2, (uz + np.uint32(1)) * _PI2)
                for a in range(2):
                    for b in range(2):
                        hxy = hx[a] ^ hy[b]
                        for c in range(2):
                            h = ((hxy ^ hz[c]) & _MASK).astype(jnp.int32)
                            blk = 4 * a + 2 * b + c
                            e2 = 2 * h
                            off = blk * (2 * _C) + 2 * (g * 16)
                            plsc.store_scatter(idxv, [off + pos_e, ], e2)
                            plsc.store_scatter(idxv, [off + pos_e + 1, ], e2 + 1)
                return c2

            lax.fori_loop(0, _C // 16, idx_body, 0)

            pltpu.async_copy(tabs.at[idxv], rowsv, sem)
            pltpu.make_async_copy(tabs.at[idxv], rowsv, sem).wait()

            def interp_body(g, c2):
                p0 = g * 8
                dupi = cp0 + p0 + halfl
                xd = plsc.load_gather(xv, [dupi])
                yd = plsc.load_gather(yv, [dupi])
                zd = plsc.load_gather(zv, [dupi])

                def dcoord(pd):
                    t = (pd - jnp.float32(_BB_MIN)) / cell
                    mv = t.astype(jnp.int32).astype(jnp.float32) * cell + jnp.float32(_BB_MIN)
                    den = (mv + cell) - mv
                    return (pd - mv) / den

                dx = dcoord(xd)
                dy = dcoord(yd)
                dz = dcoord(zd)
                e = []
                for j in range(8):
                    off = j * (2 * _C) + 2 * p0
                    e.append(rowsv[pl.ds(off, 16)])
                omx = jnp.float32(1.0) - dx
                c00 = e[0] * omx + e[4] * dx
                c01 = e[1] * omx + e[5] * dx
                c10 = e[2] * omx + e[6] * dx
                c11 = e[3] * omx + e[7] * dx
                omy = jnp.float32(1.0) - dy
                c0 = c00 * omy + c10 * dy
                c1 = c01 * omy + c11 * dy
                c = c0 * (jnp.float32(1.0) - dz) + c1 * dz
                plsc.store_scatter(outv, [p0 + halfl, feat], c)
                return c2

            lax.fori_loop(0, _C // 8, interp_body, 0)

            pltpu.sync_copy(outv, out_hbm.at[i, pl.ds(pbase + cp0, _C), :])
            return carry

        lax.fori_loop(0, _NCH, chunk_body, 0)

        plsc.subcore_barrier()


_launch = pl.kernel(
    _sc_body,
    out_type=jax.ShapeDtypeStruct((_L, _N, _F), jnp.float32),
    mesh=plsc.VectorSubcoreMesh(core_axis_name="c", subcore_axis_name="s"),
    compiler_params=pltpu.CompilerParams(
        needs_layout_passes=False, use_tc_tiling_on_sc=False),
    scratch_types=[
        pltpu.VMEM((_P,), jnp.float32),
        pltpu.VMEM((_P,), jnp.float32),
        pltpu.VMEM((_P,), jnp.float32),
        pltpu.VMEM((_E,), jnp.int32),
        pltpu.VMEM((_E,), jnp.float32),
        pltpu.VMEM((_C, _F), jnp.float32),
        pltpu.VMEM_SHARED((2 * _T,), jnp.float32),
        pltpu.SemaphoreType.DMA,
    ],
)


def kernel(xyz, tables):
    xyzt = xyz.T
    tab = tables.reshape(_L * _T * _F)
    out = _launch(xyzt[0], xyzt[1], xyzt[2], tab)
    return out.transpose(1, 0, 2).reshape(_N, _L * _F)


# speedup vs baseline: 10.1385x; 10.1385x over previous
import math

import jax
import jax.numpy as jnp
import numpy as np
from jax import lax
from jax.experimental import pallas as pl
from jax.experimental.pallas import tpu as pltpu
from jax.experimental.pallas import tpu_sc as plsc

_L = 16
_T = 2 ** 19
_F = 2
_N_MIN = 16
_N_MAX = 2048
_BB_MIN = -1.0
_GROWTH = math.exp((math.log(_N_MAX) - math.log(_N_MIN)) / (_L - 1))
_RES = [int(math.floor(_N_MIN * (_GROWTH ** i))) for i in range(_L)]
_CELL = [np.float32(2.0 / r) for r in _RES]
_PI1 = np.uint32(2654435761)
_PI2 = np.uint32(805459861)
_MASK = np.uint32(_T - 1)

_N = 262144
_NW = 32
_P = _N // _NW
_C = 512
_NCH = _P // _C
_E = 2 * 8 * _C


def _sc_body(x_hbm, y_hbm, z_hbm, tab_hbm, out_hbm,
             xv, yv, zv, idxv, rowsv, outv, tabs, sem):
    sid = lax.axis_index("s")
    wid = sid * 2 + lax.axis_index("c")
    lane = lax.iota(jnp.int32, 16)
    halfl = lax.shift_right_logical(lane, 1)
    feat = lane & 1
    pos_e = 2 * lane

    pbase = wid * _P
    pltpu.sync_copy(x_hbm.at[pl.ds(pbase, _P)], xv)
    pltpu.sync_copy(y_hbm.at[pl.ds(pbase, _P)], yv)
    pltpu.sync_copy(z_hbm.at[pl.ds(pbase, _P)], zv)

    for i in range(_L):
        cell = _CELL[i]

        @pl.when(sid == 0)
        def _():
            pltpu.sync_copy(tab_hbm.at[pl.ds(i * (2 * _T), 2 * _T)], tabs)

        plsc.subcore_barrier()

        def chunk_body(ch, carry):
            cp0 = ch * _C

            def idx_body(g, c2):
                col = cp0 + g * 16
                px = xv[pl.ds(col, 16)]
                py = yv[pl.ds(col, 16)]
                pz = zv[pl.ds(col, 16)]
                ux = ((px - jnp.float32(_BB_MIN)) / cell).astype(jnp.int32).astype(jnp.uint32)
                uy = ((py - jnp.float32(_BB_MIN)) / cell).astype(jnp.int32).astype(jnp.uint32)
                uz = ((pz - jnp.float32(_BB_MIN)) / cell).astype(jnp.int32).astype(jnp.uint32)
                hx = (ux, ux + np.uint32(1))
                hy = (uy * _PI1, (uy + np.uint32(1)) * _PI1)
                hz = (uz * _PI2, (uz + np.uint32(1)) * _PI2)
                for a in range(2):
                    for b in range(2):
                        hxy = hx[a] ^ hy[b]
                        for c in range(2):
                            h = ((hxy ^ hz[c]) & _MASK).astype(jnp.int32)
                            blk = 4 * a + 2 * b + c
                            a0 = ((h >> 7) << 8) + (h & 127)
                            off = blk * (2 * _C) + 2 * (g * 16)
                            plsc.store_scatter(idxv, [off + pos_e, ], a0)
                            plsc.store_scatter(idxv, [off + pos_e + 1, ], a0 + 128)
                return c2

            lax.fori_loop(0, _C // 16, idx_body, 0)

            pltpu.async_copy(tabs.at[idxv], rowsv, sem)
            pltpu.make_async_copy(tabs.at[idxv], rowsv, sem).wait()

            def interp_body(g, c2):
                p0 = g * 8
                opos = ((p0 >> 7) << 8) + (p0 & 127)
                dupi = cp0 + p0 + halfl
                xd = plsc.load_gather(xv, [dupi])
                yd = plsc.load_gather(yv, [dupi])
                zd = plsc.load_gather(zv, [dupi])

                def dcoord(pd):
                    t = (pd - jnp.float32(_BB_MIN)) / cell
                    mv = t.astype(jnp.int32).astype(jnp.float32) * cell + jnp.float32(_BB_MIN)
                    den = (mv + cell) - mv
                    return (pd - mv) / den

                dx = dcoord(xd)
                dy = dcoord(yd)
                dz = dcoord(zd)
                e = []
                for j in range(8):
                    off = j * (2 * _C) + 2 * p0
                    e.append(rowsv[pl.ds(off, 16)])
                omx = jnp.float32(1.0) - dx
                c00 = e[0] * omx + e[4] * dx
                c01 = e[1] * omx + e[5] * dx
                c10 = e[2] * omx + e[6] * dx
                c11 = e[3] * omx + e[7] * dx
                omy = jnp.float32(1.0) - dy
                c0 = c00 * omy + c10 * dy
                c1 = c01 * omy + c11 * dy
                c = c0 * (jnp.float32(1.0) - dz) + c1 * dz
                plsc.store_scatter(outv, [opos + (feat << 7) + halfl, ], c)
                return c2

            lax.fori_loop(0, _C // 8, interp_body, 0)

            cb = i // 4
            r = 2 * (i % 4)
            pbg = (pbase + cp0) // 128
            for pb in range(_C // 128):
                off = ((cb * (_N // 128) + pbg + pb) * 8 + r) * 128
                pltpu.sync_copy(outv.at[pl.ds(pb * 256, 256)],
                                out_hbm.at[pl.ds(off, 256)])
            return carry

        lax.fori_loop(0, _NCH, chunk_body, 0)

        plsc.subcore_barrier()


_launch = pl.kernel(
    _sc_body,
    out_type=jax.ShapeDtypeStruct((_N * 2 * _L,), jnp.float32),
    mesh=plsc.VectorSubcoreMesh(core_axis_name="c", subcore_axis_name="s"),
    compiler_params=pltpu.CompilerParams(
        needs_layout_passes=False, use_tc_tiling_on_sc=False),
    scratch_types=[
        pltpu.VMEM((_P,), jnp.float32),
        pltpu.VMEM((_P,), jnp.float32),
        pltpu.VMEM((_P,), jnp.float32),
        pltpu.VMEM((_E,), jnp.int32),
        pltpu.VMEM((_E,), jnp.float32),
        pltpu.VMEM((_C * _F,), jnp.float32),
        pltpu.VMEM_SHARED((2 * _T,), jnp.float32),
        pltpu.SemaphoreType.DMA,
    ],
)


def kernel(xyz, tables):
    xyzt = xyz.T
    tab = tables.reshape(_L, _T // 128, 128, _F).transpose(0, 1, 3, 2).reshape(-1)
    out = _launch(xyzt[0], xyzt[1], xyzt[2], tab)
    return (out.reshape(4, _N // 128, 8, 128)
            .transpose(1, 3, 0, 2).reshape(_N, _L * _F))


# speedup vs baseline: 11.4495x; 1.1293x over previous
import math

import jax
import jax.numpy as jnp
import numpy as np
from jax import lax
from jax.experimental import pallas as pl
from jax.experimental.pallas import tpu as pltpu
from jax.experimental.pallas import tpu_sc as plsc

_L = 16
_T = 2 ** 19
_F = 2
_N_MIN = 16
_N_MAX = 2048
_BB_MIN = -1.0
_GROWTH = math.exp((math.log(_N_MAX) - math.log(_N_MIN)) / (_L - 1))
_RES = [int(math.floor(_N_MIN * (_GROWTH ** i))) for i in range(_L)]
_CELL = [np.float32(2.0 / r) for r in _RES]
_INV = [np.float32(1.0) / c for c in _CELL]
_PI1 = np.uint32(2654435761)
_PI2 = np.uint32(805459861)
_MASK = np.uint32(_T - 1)

_N = 262144
_NW = 32
_P = _N // _NW
_C = 512
_NCH = _P // _C
_E = 2 * 8 * _C


def _sc_body(x_hbm, y_hbm, z_hbm, tab_hbm, out_hbm,
             xv, yv, zv, idxv, rowsv, outv, tabs, sem, osem):
    sid = lax.axis_index("s")
    wid = sid * 2 + lax.axis_index("c")
    lane = lax.iota(jnp.int32, 16)
    halfl = lax.shift_right_logical(lane, 1)
    feat = lane & 1
    pos_e = 2 * lane

    pbase = wid * _P
    pltpu.sync_copy(x_hbm.at[pl.ds(pbase, _P)], xv)
    pltpu.sync_copy(y_hbm.at[pl.ds(pbase, _P)], yv)
    pltpu.sync_copy(z_hbm.at[pl.ds(pbase, _P)], zv)

    for i in range(_L):
        cell = _CELL[i]
        inv_cell = _INV[i]

        @pl.when(sid == 0)
        def _():
            pltpu.sync_copy(tab_hbm.at[pl.ds(i * (2 * _T), 2 * _T)], tabs)

        plsc.subcore_barrier()

        def chunk_body(ch, carry):
            cp0 = ch * _C

            def idx_body(g, c2):
                col = cp0 + g * 16
                px = xv[pl.ds(col, 16)]
                py = yv[pl.ds(col, 16)]
                pz = zv[pl.ds(col, 16)]
                ux = ((px - jnp.float32(_BB_MIN)) / cell).astype(jnp.int32).astype(jnp.uint32)
                uy = ((py - jnp.float32(_BB_MIN)) / cell).astype(jnp.int32).astype(jnp.uint32)
                uz = ((pz - jnp.float32(_BB_MIN)) / cell).astype(jnp.int32).astype(jnp.uint32)
                hx = (ux, ux + np.uint32(1))
                hy = (uy * _PI1, (uy + np.uint32(1)) * _PI1)
                hz = (uz * _PI2, (uz + np.uint32(1)) * _PI2)
                for a in range(2):
                    for b in range(2):
                        hxy = hx[a] ^ hy[b]
                        for c in range(2):
                            h = ((hxy ^ hz[c]) & _MASK).astype(jnp.int32)
                            blk = 4 * a + 2 * b + c
                            a0 = ((h >> 7) << 8) + (h & 127)
                            off = blk * (2 * _C) + 2 * (g * 16)
                            plsc.store_scatter(idxv, [off + pos_e, ], a0)
                            plsc.store_scatter(idxv, [off + pos_e + 1, ], a0 + 128)
                return c2

            lax.fori_loop(0, _C // 16, idx_body, 0)

            pltpu.async_copy(tabs.at[idxv], rowsv, sem)
            pltpu.make_async_copy(tabs.at[idxv], rowsv, sem).wait()

            def interp_body(g, c2):
                p0 = g * 8
                opos = ((p0 >> 7) << 8) + (p0 & 127)
                dupi = cp0 + p0 + halfl
                xd = plsc.load_gather(xv, [dupi])
                yd = plsc.load_gather(yv, [dupi])
                zd = plsc.load_gather(zv, [dupi])

                def dcoord(pd):
                    t = (pd - jnp.float32(_BB_MIN)) / cell
                    mv = t.astype(jnp.int32).astype(jnp.float32) * cell + jnp.float32(_BB_MIN)
                    return (pd - mv) * inv_cell

                dx = dcoord(xd)
                dy = dcoord(yd)
                dz = dcoord(zd)
                e = []
                for j in range(8):
                    off = j * (2 * _C) + 2 * p0
                    e.append(rowsv[pl.ds(off, 16)])
                omx = jnp.float32(1.0) - dx
                c00 = e[0] * omx + e[4] * dx
                c01 = e[1] * omx + e[5] * dx
                c10 = e[2] * omx + e[6] * dx
                c11 = e[3] * omx + e[7] * dx
                omy = jnp.float32(1.0) - dy
                c0 = c00 * omy + c10 * dy
                c1 = c01 * omy + c11 * dy
                c = c0 * (jnp.float32(1.0) - dz) + c1 * dz
                plsc.store_scatter(outv, [opos + (feat << 7) + halfl, ], c)
                return c2

            @pl.when(ch > 0)
            def _():
                pltpu.make_async_copy(outv, out_hbm.at[pl.ds(0, _C * _F)],
                                      osem).wait()

            lax.fori_loop(0, _C // 8, interp_body, 0)

            cb = i // 4
            r = 2 * (i % 4)
            pbg = (pbase + cp0) // 128
            for pb in range(_C // 128):
                off = ((cb * (_N // 128) + pbg + pb) * 8 + r) * 128
                pltpu.async_copy(outv.at[pl.ds(pb * 256, 256)],
                                 out_hbm.at[pl.ds(off, 256)], osem)
            return carry

        lax.fori_loop(0, _NCH, chunk_body, 0)
        pltpu.make_async_copy(outv, out_hbm.at[pl.ds(0, _C * _F)], osem).wait()

        plsc.subcore_barrier()


_launch = pl.kernel(
    _sc_body,
    out_type=jax.ShapeDtypeStruct((_N * 2 * _L,), jnp.float32),
    mesh=plsc.VectorSubcoreMesh(core_axis_name="c", subcore_axis_name="s"),
    compiler_params=pltpu.CompilerParams(
        needs_layout_passes=False, use_tc_tiling_on_sc=False),
    scratch_types=[
        pltpu.VMEM((_P,), jnp.float32),
        pltpu.VMEM((_P,), jnp.float32),
        pltpu.VMEM((_P,), jnp.float32),
        pltpu.VMEM((_E,), jnp.int32),
        pltpu.VMEM((_E,), jnp.float32),
        pltpu.VMEM((_C * _F,), jnp.float32),
        pltpu.VMEM_SHARED((2 * _T,), jnp.float32),
        pltpu.SemaphoreType.DMA,
        pltpu.SemaphoreType.DMA,
    ],
)


def kernel(xyz, tables):
    xyzt = xyz.T
    tab = tables.reshape(_L, _T // 128, 128, _F).transpose(0, 1, 3, 2).reshape(-1)
    out = _launch(xyzt[0], xyzt[1], xyzt[2], tab)
    return (out.reshape(4, _N // 128, 8, 128)
            .transpose(1, 3, 0, 2).reshape(_N, _L * _F))


# speedup vs baseline: 16.5873x; 1.4487x over previous
import math

import jax
import jax.numpy as jnp
import numpy as np
from jax import lax
from jax.experimental import pallas as pl
from jax.experimental.pallas import tpu as pltpu
from jax.experimental.pallas import tpu_sc as plsc

_L = 16
_T = 2 ** 19
_F = 2
_N_MIN = 16
_N_MAX = 2048
_BB_MIN = -1.0
_GROWTH = math.exp((math.log(_N_MAX) - math.log(_N_MIN)) / (_L - 1))
_RES = [int(math.floor(_N_MIN * (_GROWTH ** i))) for i in range(_L)]
_CELL = [np.float32(2.0 / r) for r in _RES]
_INV = [np.float32(1.0) / c for c in _CELL]
_PI1 = np.uint32(2654435761)
_PI2 = np.uint32(805459861)
_MASK = np.uint32(_T - 1)

_N = 262144
_NW = 32
_P = _N // _NW
_C = 256
_NCH = _P // _C
_E = 2 * 8 * _C


def _sc_body(x_hbm, y_hbm, z_hbm, tab_hbm, out_hbm,
             xv, yv, zv, idxv, rowsv, outv, tabs, sem, osem):
    sid = lax.axis_index("s")
    wid = sid * 2 + lax.axis_index("c")
    lane = lax.iota(jnp.int32, 16)
    halfl = lax.shift_right_logical(lane, 1)
    feat = lane & 1
    pos_e = 2 * lane

    pbase = wid * _P
    pltpu.sync_copy(x_hbm.at[pl.ds(pbase, _P)], xv)
    pltpu.sync_copy(y_hbm.at[pl.ds(pbase, _P)], yv)
    pltpu.sync_copy(z_hbm.at[pl.ds(pbase, _P)], zv)

    for i in range(_L):
        cell = _CELL[i]
        inv_cell = _INV[i]

        @pl.when(sid == 0)
        def _():
            pltpu.sync_copy(tab_hbm.at[pl.ds(i * (2 * _T), 2 * _T)], tabs)

        plsc.subcore_barrier()

        def build_fire(ch):
            poff = (ch & 1) * _E

            def idx_body(g, c2):
                col = ch * _C + g * 16
                px = xv[pl.ds(col, 16)]
                py = yv[pl.ds(col, 16)]
                pz = zv[pl.ds(col, 16)]
                ux = ((px - jnp.float32(_BB_MIN)) / cell).astype(jnp.int32).astype(jnp.uint32)
                uy = ((py - jnp.float32(_BB_MIN)) / cell).astype(jnp.int32).astype(jnp.uint32)
                uz = ((pz - jnp.float32(_BB_MIN)) / cell).astype(jnp.int32).astype(jnp.uint32)
                hx = (ux, ux + np.uint32(1))
                hy = (uy * _PI1, (uy + np.uint32(1)) * _PI1)
                hz = (uz * _PI2, (uz + np.uint32(1)) * _PI2)
                for a in range(2):
                    for b in range(2):
                        hxy = hx[a] ^ hy[b]
                        for c in range(2):
                            h = ((hxy ^ hz[c]) & _MASK).astype(jnp.int32)
                            blk = 4 * a + 2 * b + c
                            a0 = ((h >> 7) << 8) + (h & 127)
                            off = poff + blk * (2 * _C) + 2 * (g * 16)
                            plsc.store_scatter(idxv, [off + pos_e, ], a0)
                            plsc.store_scatter(idxv, [off + pos_e + 1, ], a0 + 128)
                return c2

            lax.fori_loop(0, _C // 16, idx_body, 0)
            pltpu.async_copy(tabs.at[idxv.at[pl.ds(poff, _E)]],
                             rowsv.at[pl.ds(poff, _E)], sem)

        def drain_interp(ch):
            poff = (ch & 1) * _E
            ooff = (ch & 1) * (_C * _F)
            pltpu.make_async_copy(tabs.at[idxv.at[pl.ds(poff, _E)]],
                                  rowsv.at[pl.ds(poff, _E)], sem).wait()

            @pl.when(ch >= 2)
            def _():
                pltpu.make_async_copy(outv.at[pl.ds(ooff, _C * _F)],
                                      out_hbm.at[pl.ds(0, _C * _F)], osem).wait()

            def interp_body(g, c2):
                p0 = g * 8
                opos = ooff + ((p0 >> 7) << 8) + (p0 & 127)
                dupi = ch * _C + p0 + halfl
                xd = plsc.load_gather(xv, [dupi])
                yd = plsc.load_gather(yv, [dupi])
                zd = plsc.load_gather(zv, [dupi])

                def dcoord(pd):
                    t = (pd - jnp.float32(_BB_MIN)) / cell
                    mv = t.astype(jnp.int32).astype(jnp.float32) * cell + jnp.float32(_BB_MIN)
                    return (pd - mv) * inv_cell

                dx = dcoord(xd)
                dy = dcoord(yd)
                dz = dcoord(zd)
                e = []
                for j in range(8):
                    off = poff + j * (2 * _C) + 2 * p0
                    e.append(rowsv[pl.ds(off, 16)])
                omx = jnp.float32(1.0) - dx
                c00 = e[0] * omx + e[4] * dx
                c01 = e[1] * omx + e[5] * dx
                c10 = e[2] * omx + e[6] * dx
                c11 = e[3] * omx + e[7] * dx
                omy = jnp.float32(1.0) - dy
                c0 = c00 * omy + c10 * dy
                c1 = c01 * omy + c11 * dy
                c = c0 * (jnp.float32(1.0) - dz) + c1 * dz
                plsc.store_scatter(outv, [opos + (feat << 7) + halfl, ], c)
                return c2

            lax.fori_loop(0, _C // 8, interp_body, 0)

            cb = i // 4
            r = 2 * (i % 4)
            pbg = (pbase + ch * _C) // 128
            for pb in range(_C // 128):
                off = ((cb * (_N // 128) + pbg + pb) * 8 + r) * 128
                pltpu.async_copy(outv.at[pl.ds(ooff + pb * 256, 256)],
                                 out_hbm.at[pl.ds(off, 256)], osem)

        def phase_body(ph, carry):
            @pl.when(ph < _NCH)
            def _():
                build_fire(ph)

            @pl.when(ph > 0)
            def _():
                drain_interp(ph - 1)

            return carry

        lax.fori_loop(0, _NCH + 1, phase_body, 0)
        pltpu.make_async_copy(outv, out_hbm.at[pl.ds(0, 2 * _C * _F)], osem).wait()

        plsc.subcore_barrier()


_launch = pl.kernel(
    _sc_body,
    out_type=jax.ShapeDtypeStruct((_N * 2 * _L,), jnp.float32),
    mesh=plsc.VectorSubcoreMesh(core_axis_name="c", subcore_axis_name="s"),
    compiler_params=pltpu.CompilerParams(
        needs_layout_passes=False, use_tc_tiling_on_sc=False),
    scratch_types=[
        pltpu.VMEM((_P,), jnp.float32),
        pltpu.VMEM((_P,), jnp.float32),
        pltpu.VMEM((_P,), jnp.float32),
        pltpu.VMEM((2 * _E,), jnp.int32),
        pltpu.VMEM((2 * _E,), jnp.float32),
        pltpu.VMEM((2 * _C * _F,), jnp.float32),
        pltpu.VMEM_SHARED((2 * _T,), jnp.float32),
        pltpu.SemaphoreType.DMA,
        pltpu.SemaphoreType.DMA,
    ],
)


def kernel(xyz, tables):
    xyzt = xyz.T
    tab = tables.reshape(_L, _T // 128, 128, _F).transpose(0, 1, 3, 2).reshape(-1)
    out = _launch(xyzt[0], xyzt[1], xyzt[2], tab)
    return (out.reshape(4, _N // 128, 8, 128)
            .transpose(1, 3, 0, 2).reshape(_N, _L * _F))
